# baseline (speedup 1.0000x reference)
import jax
import jax.numpy as jnp
from jax import lax
from jax.experimental import pallas as pl
from jax.experimental.pallas import tpu as pltpu
from jax.experimental.pallas import tpu_sc as plsc

NC, NS, L = 2, 16, 16
NW = NC * NS

B = 16384
F = 26
CHUNK = 128
BPW = B // NW
C = BPW // CHUNK


def _sc_body(xw_hbm, table_hbm, bias_hbm, out_hbm, idx_v, vals_v, bias_v,
             out_v, sem):
    wid = lax.axis_index("s") * NC + lax.axis_index("c")

    pltpu.sync_copy(xw_hbm.at[wid], idx_v)
    pltpu.sync_copy(bias_hbm, bias_v)

    def gather_one(i, carry):
        f = i // C
        c = lax.rem(i, C)
        pltpu.async_copy(table_hbm.at[idx_v.at[f, c]], vals_v.at[f, c],
                         sem).wait()
        return carry

    lax.fori_loop(0, F * C, gather_one, 0)

    bias_vec = bias_v[...]

    def reduce_one(g, carry):
        c = g // (CHUNK // L)
        o = lax.rem(g, CHUNK // L) * L
        acc = bias_vec
        for f in range(F):
            acc = acc + vals_v[f, c, pl.ds(o, L)]
        out_v[pl.ds(g * L, L)] = acc
        return carry

    lax.fori_loop(0, BPW // L, reduce_one, 0)
    pltpu.sync_copy(out_v, out_hbm.at[pl.ds(wid * BPW, BPW)])


def kernel(X, table, bias):
    Xw = (
        X.astype(jnp.int32)
        .reshape(NW, BPW, F)
        .transpose(0, 2, 1)
        .reshape(NW, F, C, CHUNK)
    )
    table1 = table.reshape(-1)
    bias16 = jnp.broadcast_to(bias.astype(jnp.float32), (L,))

    mesh = plsc.VectorSubcoreMesh(core_axis_name="c", subcore_axis_name="s")
    out = pl.kernel(
        _sc_body,
        out_type=jax.ShapeDtypeStruct((B,), jnp.float32),
        mesh=mesh,
        scratch_types=[
            pltpu.VMEM((F, C, CHUNK), jnp.int32),
            pltpu.VMEM((F, C, CHUNK), jnp.float32),
            pltpu.VMEM((L,), jnp.float32),
            pltpu.VMEM((BPW,), jnp.float32),
            pltpu.SemaphoreType.DMA,
        ],
    )(Xw, table1, bias16)
    return out.reshape(B, 1)


# speedup vs baseline: 1.7382x; 1.7382x over previous
import jax
import jax.numpy as jnp
from jax import lax
from jax.experimental import pallas as pl
from jax.experimental.pallas import tpu as pltpu
from jax.experimental.pallas import tpu_sc as plsc

NC, NS, L = 2, 16, 16
NW = NC * NS

B = 16384
F = 26
BPW = B // NW
IPW = F * BPW


def _sc_body(xw_hbm, table_hbm, bias_hbm, out_hbm, idx_v, vals_v, bias_v,
             out_v, sem):
    wid = lax.axis_index("s") * NC + lax.axis_index("c")

    pltpu.sync_copy(xw_hbm.at[wid], idx_v)
    pltpu.sync_copy(bias_hbm, bias_v)

    pltpu.async_copy(table_hbm.at[idx_v], vals_v, sem).wait()

    bias_vec = bias_v[...]

    def reduce_one(g, carry):
        o = g * L
        acc = bias_vec
        for f in range(F):
            acc = acc + vals_v[pl.ds(f * BPW + o, L)]
        out_v[pl.ds(o, L)] = acc
        return carry

    lax.fori_loop(0, BPW // L, reduce_one, 0)
    pltpu.sync_copy(out_v, out_hbm.at[pl.ds(wid * BPW, BPW)])


def kernel(X, table, bias):
    Xw = (
        X.astype(jnp.int32)
        .reshape(NW, BPW, F)
        .transpose(0, 2, 1)
        .reshape(NW, IPW)
    )
    table1 = table.reshape(-1)
    bias16 = jnp.broadcast_to(bias.astype(jnp.float32), (L,))

    mesh = plsc.VectorSubcoreMesh(core_axis_name="c", subcore_axis_name="s")
    out = pl.kernel(
        _sc_body,
        out_type=jax.ShapeDtypeStruct((B,), jnp.float32),
        mesh=mesh,
        scratch_types=[
            pltpu.VMEM((IPW,), jnp.int32),
            pltpu.VMEM((IPW,), jnp.float32),
            pltpu.VMEM((L,), jnp.float32),
            pltpu.VMEM((BPW,), jnp.float32),
            pltpu.SemaphoreType.DMA,
        ],
    )(Xw, table1, bias16)
    return out.reshape(B, 1)


# speedup vs baseline: 3.8811x; 2.2328x over previous
import jax
import jax.numpy as jnp
from jax import lax
from jax.experimental import pallas as pl
from jax.experimental.pallas import tpu as pltpu
from jax.experimental.pallas import tpu_sc as plsc

NC, NS, L = 2, 16, 16
NW = NC * NS

B = 16384
F = 26
BPW = B // NW
IPW = F * BPW


def _sc_body(xt_hbm, tablet_hbm, bias_hbm, out_hbm, idx_v, vals_v, bias_v,
             out_v, sem, sem2):
    wid = lax.axis_index("s") * NC + lax.axis_index("c")
    base = wid * BPW

    for f in range(F):
        pltpu.async_copy(xt_hbm.at[f, pl.ds(base, BPW)],
                         idx_v.at[pl.ds(f * BPW, BPW)], sem2)
    pltpu.sync_copy(bias_hbm, bias_v)
    for f in range(F):
        pltpu.make_async_copy(xt_hbm.at[f, pl.ds(base, BPW)],
                              idx_v.at[pl.ds(f * BPW, BPW)], sem2).wait()

    table1 = tablet_hbm.at[0]
    pltpu.async_copy(table1.at[idx_v], vals_v, sem).wait()

    bias_vec = bias_v[...]

    def reduce_one(g, carry):
        o = g * L
        acc = bias_vec
        for f in range(F):
            acc = acc + vals_v[pl.ds(f * BPW + o, L)]
        out_v[pl.ds(o, L)] = acc
        return carry

    lax.fori_loop(0, BPW // L, reduce_one, 0)
    pltpu.sync_copy(out_v, out_hbm.at[pl.ds(base, BPW)])


def kernel(X, table, bias):
    Xt = X.astype(jnp.int32).T
    tablet = table.T
    bias16 = jnp.broadcast_to(bias.astype(jnp.float32), (L,))

    mesh = plsc.VectorSubcoreMesh(core_axis_name="c", subcore_axis_name="s")
    out = pl.kernel(
        _sc_body,
        out_type=jax.ShapeDtypeStruct((B,), jnp.float32),
        mesh=mesh,
        scratch_types=[
            pltpu.VMEM((IPW,), jnp.int32),
            pltpu.VMEM((IPW,), jnp.float32),
            pltpu.VMEM((L,), jnp.float32),
            pltpu.VMEM((BPW,), jnp.float32),
            pltpu.SemaphoreType.DMA,
            pltpu.SemaphoreType.DMA,
        ],
    )(Xt, tablet, bias16)
    return out.reshape(B, 1)
